# baseline (speedup 1.0000x reference)
import functools
import math

import jax
import jax.numpy as jnp
from jax import lax
from jax.experimental import pallas as pl
from jax.experimental.pallas import tpu as pltpu
from jax.experimental.pallas import tpu_sc as plsc

NC = 2
NS = 16
B = 80
IB = 8
BN_EPS = 1e-5
ISQ = 1.0 / math.sqrt(1.0 + BN_EPS)


def _sc_mesh():
    return plsc.VectorSubcoreMesh(
        core_axis_name="c", subcore_axis_name="s", num_cores=NC, num_subcores=NS
    )


def _geometry(E):
    nt = NC * NS
    assert E % (nt * B) == 0
    return nt, E // (nt * B)


def _make_deg_kernel(N, E):
    _, nb = _geometry(E)
    nd = -(-N // 16) * 16
    assert nd == N

    @functools.partial(
        pl.kernel,
        out_type=jax.ShapeDtypeStruct((NC, nd), jnp.float32),
        mesh=_sc_mesh(),
        scratch_types=[
            pltpu.VMEM_SHARED((nd,), jnp.float32),
            pltpu.VMEM((nb, B), jnp.int32),
            pltpu.VMEM((B,), jnp.float32),
            pltpu.VMEM((nd,), jnp.float32),
        ],
    )
    def deg_kernel(dst3_hbm, deg_hbm, deg_sh, idx_v, ones_v, zbuf):
        cid = lax.axis_index("c")
        sid = lax.axis_index("s")
        tid = cid * NS + sid

        for off in range(0, B, 16):
            ones_v[pl.ds(off, 16)] = jnp.ones((16,), jnp.float32)

        @pl.when(sid == 0)
        def _():
            def zstep(i, c):
                zbuf[pl.ds(i * 16, 16)] = jnp.zeros((16,), jnp.float32)
                return c
            lax.fori_loop(0, nd // 16, zstep, 0)
            pltpu.sync_copy(zbuf, deg_sh)

        plsc.subcore_barrier()

        pltpu.sync_copy(dst3_hbm.at[tid], idx_v)

        def step(i, c):
            pltpu.sync_copy(ones_v, deg_sh.at[idx_v.at[i]], add=True)
            return c
        lax.fori_loop(0, nb, step, 0)

        plsc.subcore_barrier()

        @pl.when(sid == 0)
        def _():
            pltpu.sync_copy(deg_sh, deg_hbm.at[cid])

    return deg_kernel


def _make_scatter_kernel(N, H, E):
    nt, nb = _geometry(E)
    ept = nb * B
    rpt = (N // NS) // 8 * 8
    tail = N - NS * rpt
    assert nb % 2 == 1 and tail % 8 == 0 and ept % 8 == 0

    @functools.partial(
        pl.kernel,
        out_type=jax.ShapeDtypeStruct((NC, N, H), jnp.float32),
        mesh=_sc_mesh(),
        scratch_types=[
            pltpu.VMEM_SHARED((N, H), jnp.float32),
            pltpu.VMEM((ept,), jnp.int32),
            pltpu.VMEM((nb, B), jnp.int32),
            pltpu.VMEM((B, H), jnp.float32),
            pltpu.VMEM((B, H), jnp.float32),
            pltpu.SemaphoreType.DMA,
            pltpu.SemaphoreType.DMA,
            pltpu.SemaphoreType.DMA,
            pltpu.SemaphoreType.DMA,
        ],
    )
    def scatter_kernel(u_hbm, src1_hbm, dst3_hbm, acc_hbm,
                       acc_sh, sidx, didx, rows0, rows1,
                       sem0, sem1, ssem0, ssem1):
        cid = lax.axis_index("c")
        sid = lax.axis_index("s")
        tid = cid * NS + sid

        pltpu.sync_copy(u_hbm.at[pl.ds(sid * rpt, rpt)],
                        acc_sh.at[pl.ds(sid * rpt, rpt)])

        @pl.when(sid == 0)
        def _():
            pltpu.sync_copy(u_hbm.at[pl.ds(NS * rpt, tail)],
                            acc_sh.at[pl.ds(NS * rpt, tail)])

        plsc.subcore_barrier()

        pltpu.sync_copy(src1_hbm.at[pl.ds(tid * ept, ept)], sidx)
        pltpu.sync_copy(dst3_hbm.at[tid], didx)

        def gather(i, buf, sem):
            off = pl.multiple_of(i * B, B)
            pltpu.async_copy(u_hbm.at[sidx.at[pl.ds(off, B)]], buf, sem)

        def drain(buf, sem):
            pltpu.make_async_copy(u_hbm.at[pl.ds(0, B)], buf, sem).wait()

        def sdrain(buf, sem):
            pltpu.make_async_copy(buf, acc_sh.at[pl.ds(0, B)], sem).wait()

        gather(0, rows0, sem0)
        gather(1, rows1, sem1)

        def step(j, c):
            i = 2 * j
            drain(rows0, sem0)
            pltpu.async_copy(rows0, acc_sh.at[didx.at[i]], ssem0, add=True)
            drain(rows1, sem1)
            pltpu.async_copy(rows1, acc_sh.at[didx.at[i + 1]], ssem1,
                             add=True)
            sdrain(rows0, ssem0)

            @pl.when(i + 2 < nb)
            def _():
                gather(i + 2, rows0, sem0)

            sdrain(rows1, ssem1)

            @pl.when(i + 3 < nb)
            def _():
                gather(i + 3, rows1, sem1)
            return c
        lax.fori_loop(0, (nb - 1) // 2, step, 0)

        drain(rows0, sem0)
        pltpu.sync_copy(rows0, acc_sh.at[didx.at[nb - 1]], add=True)

        plsc.subcore_barrier()

        pltpu.sync_copy(acc_sh.at[pl.ds(sid * rpt, rpt)],
                        acc_hbm.at[cid, pl.ds(sid * rpt, rpt)])

        @pl.when(sid == 0)
        def _():
            pltpu.sync_copy(acc_sh.at[pl.ds(NS * rpt, tail)],
                            acc_hbm.at[cid, pl.ds(NS * rpt, tail)])

    return scatter_kernel


def _tc_prep(x, d0, d1, W):
    N, D = x.shape
    H = W.shape[1]
    R = 1000

    def body(x_ref, d0_ref, d1_ref, w_ref, u_ref, dis_ref):
        deg = d0_ref[...] + d1_ref[...] + 1.0
        dis = lax.rsqrt(deg)
        h = jnp.dot(x_ref[...], w_ref[...], preferred_element_type=jnp.float32)
        u_ref[...] = h * dis
        dis_ref[...] = dis

    return pl.pallas_call(
        body,
        grid=(N // R,),
        in_specs=[
            pl.BlockSpec((R, D), lambda i: (i, 0)),
            pl.BlockSpec((R, 1), lambda i: (i, 0)),
            pl.BlockSpec((R, 1), lambda i: (i, 0)),
            pl.BlockSpec((D, H), lambda i: (0, 0)),
        ],
        out_specs=[
            pl.BlockSpec((R, H), lambda i: (i, 0)),
            pl.BlockSpec((R, 1), lambda i: (i, 0)),
        ],
        out_shape=[
            jax.ShapeDtypeStruct((N, H), jnp.float32),
            jax.ShapeDtypeStruct((N, 1), jnp.float32),
        ],
    )(x, d0, d1, W)


def _tc_mid(a0, a1, up, dis, W, b, g, be):
    N, H = a0.shape
    R = 1000

    def body(a0_ref, a1_ref, up_ref, dis_ref, w_ref, b_ref, g_ref, be_ref,
             u_ref):
        dis_v = dis_ref[...]
        s = (a0_ref[...] + a1_ref[...] - up_ref[...]) * dis_v + b_ref[...]
        xn = jnp.maximum(s * (g_ref[...] * ISQ) + be_ref[...], 0.0)
        u_ref[...] = jnp.dot(
            xn, w_ref[...], preferred_element_type=jnp.float32) * dis_v

    return pl.pallas_call(
        body,
        grid=(N // R,),
        in_specs=[
            pl.BlockSpec((R, H), lambda i: (i, 0)),
            pl.BlockSpec((R, H), lambda i: (i, 0)),
            pl.BlockSpec((R, H), lambda i: (i, 0)),
            pl.BlockSpec((R, 1), lambda i: (i, 0)),
            pl.BlockSpec((H, H), lambda i: (0, 0)),
            pl.BlockSpec((1, H), lambda i: (0, 0)),
            pl.BlockSpec((1, H), lambda i: (0, 0)),
            pl.BlockSpec((1, H), lambda i: (0, 0)),
        ],
        out_specs=pl.BlockSpec((R, H), lambda i: (i, 0)),
        out_shape=jax.ShapeDtypeStruct((N, H), jnp.float32),
    )(a0, a1, up, dis, W, b, g, be)


def _tc_final(a0, a1, up, dis, b, g, be):
    N, H = a0.shape
    R = 1000

    def body(a0_ref, a1_ref, up_ref, dis_ref, b_ref, g_ref, be_ref,
             h_ref, m_ref):
        i = pl.program_id(0)
        s = (a0_ref[...] + a1_ref[...] - up_ref[...]) * dis_ref[...] + b_ref[...]
        xn = jnp.maximum(s * (g_ref[...] * ISQ) + be_ref[...], 0.0)
        h_ref[...] = xn
        part = jnp.sum(xn, axis=0, keepdims=True) * (1.0 / N)

        @pl.when(i == 0)
        def _():
            m_ref[...] = part

        @pl.when(i > 0)
        def _():
            m_ref[...] += part

    return pl.pallas_call(
        body,
        grid=(N // R,),
        in_specs=[
            pl.BlockSpec((R, H), lambda i: (i, 0)),
            pl.BlockSpec((R, H), lambda i: (i, 0)),
            pl.BlockSpec((R, H), lambda i: (i, 0)),
            pl.BlockSpec((R, 1), lambda i: (i, 0)),
            pl.BlockSpec((1, H), lambda i: (0, 0)),
            pl.BlockSpec((1, H), lambda i: (0, 0)),
            pl.BlockSpec((1, H), lambda i: (0, 0)),
        ],
        out_specs=[
            pl.BlockSpec((R, H), lambda i: (i, 0)),
            pl.BlockSpec((1, H), lambda i: (0, 0)),
        ],
        out_shape=[
            jax.ShapeDtypeStruct((N, H), jnp.float32),
            jax.ShapeDtypeStruct((1, H), jnp.float32),
        ],
    )(a0, a1, up, dis, b, g, be)


def kernel(x, edge_index, W1, b1, g1, be1, W2, b2, g2, be2, W3, b3, g3, be3):
    N, D = x.shape
    H = W1.shape[1]
    E = edge_index.shape[1]

    nt, nb = _geometry(E)
    src1 = edge_index[0]
    dst3 = edge_index[1].reshape(nt, nb, B)

    deg_parts = _make_deg_kernel(N, E)(dst3)
    d0 = deg_parts[0, :N].reshape(N, 1)
    d1 = deg_parts[1, :N].reshape(N, 1)

    u1, dis = _tc_prep(x, d0, d1, W1)

    scatter = _make_scatter_kernel(N, H, E)
    acc = scatter(u1, src1, dst3)
    u2 = _tc_mid(acc[0], acc[1], u1, dis, W2,
                 b1.reshape(1, H), g1.reshape(1, H), be1.reshape(1, H))
    acc = scatter(u2, src1, dst3)
    u3 = _tc_mid(acc[0], acc[1], u2, dis, W3,
                 b2.reshape(1, H), g2.reshape(1, H), be2.reshape(1, H))
    acc = scatter(u3, src1, dst3)
    h, gmean = _tc_final(acc[0], acc[1], u3, dis,
                         b3.reshape(1, H), g3.reshape(1, H), be3.reshape(1, H))
    return (h, gmean)


# speedup vs baseline: 1.4175x; 1.4175x over previous
import functools
import math

import jax
import jax.numpy as jnp
from jax import lax
from jax.experimental import pallas as pl
from jax.experimental.pallas import tpu as pltpu
from jax.experimental.pallas import tpu_sc as plsc

NC = 2
NS = 16
B = 80
IB = 8
BN_EPS = 1e-5
ISQ = 1.0 / math.sqrt(1.0 + BN_EPS)


def _sc_mesh():
    return plsc.VectorSubcoreMesh(
        core_axis_name="c", subcore_axis_name="s", num_cores=NC, num_subcores=NS
    )


def _geometry(E):
    nt = NC * NS
    assert E % (nt * B) == 0
    return nt, E // (nt * B)


def _make_deg_kernel(N, E):
    _, nb = _geometry(E)
    nd = -(-N // 16) * 16
    assert nd == N

    @functools.partial(
        pl.kernel,
        out_type=jax.ShapeDtypeStruct((NC, nd), jnp.float32),
        mesh=_sc_mesh(),
        scratch_types=[
            pltpu.VMEM_SHARED((nd,), jnp.float32),
            pltpu.VMEM((nb, B), jnp.int32),
            pltpu.VMEM((B,), jnp.float32),
            pltpu.VMEM((nd,), jnp.float32),
        ],
    )
    def deg_kernel(dst3_hbm, deg_hbm, deg_sh, idx_v, ones_v, zbuf):
        cid = lax.axis_index("c")
        sid = lax.axis_index("s")
        tid = cid * NS + sid

        for off in range(0, B, 16):
            ones_v[pl.ds(off, 16)] = jnp.ones((16,), jnp.float32)

        @pl.when(sid == 0)
        def _():
            def zstep(i, c):
                zbuf[pl.ds(i * 16, 16)] = jnp.zeros((16,), jnp.float32)
                return c
            lax.fori_loop(0, nd // 16, zstep, 0)
            pltpu.sync_copy(zbuf, deg_sh)

        plsc.subcore_barrier()

        pltpu.sync_copy(dst3_hbm.at[tid], idx_v)

        def step(i, c):
            pltpu.sync_copy(ones_v, deg_sh.at[idx_v.at[i]], add=True)
            return c
        lax.fori_loop(0, nb, step, 0)

        plsc.subcore_barrier()

        @pl.when(sid == 0)
        def _():
            pltpu.sync_copy(deg_sh, deg_hbm.at[cid])

    return deg_kernel


def _make_scatter_kernel(N, H, E):
    nt, nb = _geometry(E)
    ept = nb * B
    rpt = (N // NS) // 8 * 8
    tail = N - NS * rpt
    assert nb % 2 == 1 and tail % 8 == 0 and ept % 8 == 0

    @functools.partial(
        pl.kernel,
        out_type=jax.ShapeDtypeStruct((NC, N, H), jnp.float32),
        mesh=_sc_mesh(),
        scratch_types=[
            pltpu.VMEM_SHARED((N, H), jnp.float32),
            pltpu.VMEM((ept,), jnp.int32),
            pltpu.VMEM((ept,), jnp.int32),
            pltpu.VMEM((B, H), jnp.float32),
            pltpu.VMEM((B, H), jnp.float32),
            pltpu.VMEM((B, H), jnp.float32),
            pltpu.SemaphoreType.DMA,
            pltpu.SemaphoreType.DMA,
            pltpu.SemaphoreType.DMA,
        ],
    )
    def scatter_kernel(u_hbm, src1_hbm, dst1_hbm, acc_hbm,
                       acc_sh, sidx, didx, rows0, rows1, rows2,
                       sem0, sem1, sem2):
        cid = lax.axis_index("c")
        sid = lax.axis_index("s")
        tid = cid * NS + sid

        pltpu.sync_copy(u_hbm.at[pl.ds(sid * rpt, rpt)],
                        acc_sh.at[pl.ds(sid * rpt, rpt)])

        @pl.when(sid == 0)
        def _():
            pltpu.sync_copy(u_hbm.at[pl.ds(NS * rpt, tail)],
                            acc_sh.at[pl.ds(NS * rpt, tail)])

        plsc.subcore_barrier()

        pltpu.sync_copy(src1_hbm.at[pl.ds(tid * ept, ept)], sidx)
        pltpu.sync_copy(dst1_hbm.at[pl.ds(tid * ept, ept)], didx)

        def gather(i, buf, sem):
            off = pl.multiple_of(i * B, B)
            pltpu.async_copy(u_hbm.at[sidx.at[pl.ds(off, B)]], buf, sem)

        def drain(buf, sem):
            pltpu.make_async_copy(u_hbm.at[pl.ds(0, B)], buf, sem).wait()

        def scat(i, buf):
            off = pl.multiple_of(i * B, B)
            pltpu.sync_copy(buf, acc_sh.at[didx.at[pl.ds(off, B)]], add=True)

        assert nb % 3 == 2
        gather(0, rows0, sem0)
        gather(1, rows1, sem1)

        def step(j, c):
            i = 3 * j
            gather(i + 2, rows2, sem2)
            drain(rows0, sem0)
            scat(i, rows0)
            gather(i + 3, rows0, sem0)
            drain(rows1, sem1)
            scat(i + 1, rows1)
            gather(i + 4, rows1, sem1)
            drain(rows2, sem2)
            scat(i + 2, rows2)
            return c
        lax.fori_loop(0, (nb - 2) // 3, step, 0)

        drain(rows0, sem0)
        scat(nb - 2, rows0)
        drain(rows1, sem1)
        scat(nb - 1, rows1)

        plsc.subcore_barrier()

        pltpu.sync_copy(acc_sh.at[pl.ds(sid * rpt, rpt)],
                        acc_hbm.at[cid, pl.ds(sid * rpt, rpt)])

        @pl.when(sid == 0)
        def _():
            pltpu.sync_copy(acc_sh.at[pl.ds(NS * rpt, tail)],
                            acc_hbm.at[cid, pl.ds(NS * rpt, tail)])

    return scatter_kernel


def _tc_prep(x, d0, d1, W):
    N, D = x.shape
    H = W.shape[1]
    R = 1000

    def body(x_ref, d0_ref, d1_ref, w_ref, u_ref, dis_ref):
        deg = d0_ref[...] + d1_ref[...] + 1.0
        dis = lax.rsqrt(deg)
        h = jnp.dot(x_ref[...], w_ref[...], preferred_element_type=jnp.float32)
        u_ref[...] = h * dis
        dis_ref[...] = dis

    return pl.pallas_call(
        body,
        grid=(N // R,),
        in_specs=[
            pl.BlockSpec((R, D), lambda i: (i, 0)),
            pl.BlockSpec((R, 1), lambda i: (i, 0)),
            pl.BlockSpec((R, 1), lambda i: (i, 0)),
            pl.BlockSpec((D, H), lambda i: (0, 0)),
        ],
        out_specs=[
            pl.BlockSpec((R, H), lambda i: (i, 0)),
            pl.BlockSpec((R, 1), lambda i: (i, 0)),
        ],
        out_shape=[
            jax.ShapeDtypeStruct((N, H), jnp.float32),
            jax.ShapeDtypeStruct((N, 1), jnp.float32),
        ],
    )(x, d0, d1, W)


def _tc_mid(a0, a1, up, dis, W, b, g, be):
    N, H = a0.shape
    R = 1000

    def body(a0_ref, a1_ref, up_ref, dis_ref, w_ref, b_ref, g_ref, be_ref,
             u_ref):
        dis_v = dis_ref[...]
        s = (a0_ref[...] + a1_ref[...] - up_ref[...]) * dis_v + b_ref[...]
        xn = jnp.maximum(s * (g_ref[...] * ISQ) + be_ref[...], 0.0)
        u_ref[...] = jnp.dot(
            xn, w_ref[...], preferred_element_type=jnp.float32) * dis_v

    return pl.pallas_call(
        body,
        grid=(N // R,),
        in_specs=[
            pl.BlockSpec((R, H), lambda i: (i, 0)),
            pl.BlockSpec((R, H), lambda i: (i, 0)),
            pl.BlockSpec((R, H), lambda i: (i, 0)),
            pl.BlockSpec((R, 1), lambda i: (i, 0)),
            pl.BlockSpec((H, H), lambda i: (0, 0)),
            pl.BlockSpec((1, H), lambda i: (0, 0)),
            pl.BlockSpec((1, H), lambda i: (0, 0)),
            pl.BlockSpec((1, H), lambda i: (0, 0)),
        ],
        out_specs=pl.BlockSpec((R, H), lambda i: (i, 0)),
        out_shape=jax.ShapeDtypeStruct((N, H), jnp.float32),
    )(a0, a1, up, dis, W, b, g, be)


def _tc_final(a0, a1, up, dis, b, g, be):
    N, H = a0.shape
    R = 1000

    def body(a0_ref, a1_ref, up_ref, dis_ref, b_ref, g_ref, be_ref,
             h_ref, m_ref):
        i = pl.program_id(0)
        s = (a0_ref[...] + a1_ref[...] - up_ref[...]) * dis_ref[...] + b_ref[...]
        xn = jnp.maximum(s * (g_ref[...] * ISQ) + be_ref[...], 0.0)
        h_ref[...] = xn
        part = jnp.sum(xn, axis=0, keepdims=True) * (1.0 / N)

        @pl.when(i == 0)
        def _():
            m_ref[...] = part

        @pl.when(i > 0)
        def _():
            m_ref[...] += part

    return pl.pallas_call(
        body,
        grid=(N // R,),
        in_specs=[
            pl.BlockSpec((R, H), lambda i: (i, 0)),
            pl.BlockSpec((R, H), lambda i: (i, 0)),
            pl.BlockSpec((R, H), lambda i: (i, 0)),
            pl.BlockSpec((R, 1), lambda i: (i, 0)),
            pl.BlockSpec((1, H), lambda i: (0, 0)),
            pl.BlockSpec((1, H), lambda i: (0, 0)),
            pl.BlockSpec((1, H), lambda i: (0, 0)),
        ],
        out_specs=[
            pl.BlockSpec((R, H), lambda i: (i, 0)),
            pl.BlockSpec((1, H), lambda i: (0, 0)),
        ],
        out_shape=[
            jax.ShapeDtypeStruct((N, H), jnp.float32),
            jax.ShapeDtypeStruct((1, H), jnp.float32),
        ],
    )(a0, a1, up, dis, b, g, be)


def kernel(x, edge_index, W1, b1, g1, be1, W2, b2, g2, be2, W3, b3, g3, be3):
    N, D = x.shape
    H = W1.shape[1]
    E = edge_index.shape[1]

    nt, nb = _geometry(E)
    src1 = edge_index[0]
    dst1 = edge_index[1]
    dst3 = dst1.reshape(nt, nb, B)

    deg_parts = _make_deg_kernel(N, E)(dst3)
    d0 = deg_parts[0, :N].reshape(N, 1)
    d1 = deg_parts[1, :N].reshape(N, 1)

    u1, dis = _tc_prep(x, d0, d1, W1)

    scatter = _make_scatter_kernel(N, H, E)
    acc = scatter(u1, src1, dst1)
    u2 = _tc_mid(acc[0], acc[1], u1, dis, W2,
                 b1.reshape(1, H), g1.reshape(1, H), be1.reshape(1, H))
    acc = scatter(u2, src1, dst1)
    u3 = _tc_mid(acc[0], acc[1], u2, dis, W3,
                 b2.reshape(1, H), g2.reshape(1, H), be2.reshape(1, H))
    acc = scatter(u3, src1, dst1)
    h, gmean = _tc_final(acc[0], acc[1], u3, dis,
                         b3.reshape(1, H), g3.reshape(1, H), be3.reshape(1, H))
    return (h, gmean)
